# baseline (speedup 1.0000x reference)
import dataclasses

import jax
import jax.numpy as jnp
from jax import lax
from jax.experimental import pallas as pl
from jax.experimental.pallas import tpu as pltpu
from jax.experimental.pallas import tpu_sc as plsc

VOCAB = 1000000
EMBED = 64
BATCH = 16384
HIST = 50

LANES = 16
SCALE = 8.0
NW = 32
BT = 128
TILES_PER_W = BATCH // BT // NW
HG = 5
NG = HIST // HG
EG = EMBED // 8

_mesh = plsc.VectorSubcoreMesh(core_axis_name="core", subcore_axis_name="subcore")

_cp = pltpu.CompilerParams(use_tc_tiling_on_sc=False)
if "needs_layout_passes" in pltpu.CompilerParams.__dataclass_fields__:
    _cp = dataclasses.replace(_cp, needs_layout_passes=False)


def _lookup(table2, idx):
    @pl.kernel(
        out_type=jax.ShapeDtypeStruct((HIST, EG, BATCH // BT, 8, BT), jnp.float32),
        mesh=_mesh,
        scratch_types=[
            pltpu.VMEM((BT * HIST,), jnp.int32),
            pltpu.VMEM((HIST, BT), jnp.int32),
            pltpu.VMEM((HG * BT, EMBED), jnp.float32),
            pltpu.VMEM((HG * BT, EMBED), jnp.float32),
            pltpu.VMEM((8, 8, BT), jnp.float32),
            pltpu.SemaphoreType.DMA,
            pltpu.SemaphoreType.DMA,
        ],
        compiler_params=_cp,
    )
    def kern(t_hbm, i_hbm, w_hbm, idxblk, idxt, bufa, bufb, stg, gsem, osem):
        wid = lax.axis_index("subcore") * 2 + lax.axis_index("core")
        iota = lax.iota(jnp.int32, LANES)
        p50 = iota * HIST

        def fire(buf, gi):
            for hl in range(HG):
                pltpu.async_copy(
                    t_hbm.at[idxt.at[gi * HG + hl]],
                    buf.at[pl.ds(hl * BT, BT), :],
                    gsem,
                )

        def drain(buf, gi):
            for hl in range(HG):
                pltpu.make_async_copy(
                    t_hbm.at[idxt.at[gi * HG + hl]],
                    buf.at[pl.ds(hl * BT, BT), :],
                    gsem,
                ).wait()

        @pl.loop(0, TILES_PER_W)
        def _(t):
            bt = wid * TILES_PER_W + t
            pltpu.sync_copy(i_hbm.at[pl.ds(bt * (BT * HIST), BT * HIST)], idxblk)

            @pl.loop(0, HIST)
            def _(h):
                for b16 in range(BT // LANES):
                    v = plsc.load_gather(idxblk, [p50 + (b16 * LANES * HIST + h)])
                    idxt[h, pl.ds(b16 * LANES, LANES)] = v + v

            def group(buf, gi, nbuf, prime, fire_next):
                if prime:
                    fire(buf, gi)
                if fire_next:
                    fire(nbuf, gi + 1)
                drain(buf, gi)

                @pl.loop(0, (HG * EG) // 8)
                def _(q):
                    for slot in range(8):
                        pos = q * 8 + slot
                        hl = pos // EG
                        eg = pos - hl * EG
                        h = gi * HG + hl

                        def recycle(slot=slot, h=h, eg=eg):
                            pltpu.make_async_copy(
                                stg.at[slot], w_hbm.at[h, eg, bt], osem
                            ).wait()

                        if prime:
                            pl.when(q >= 1)(recycle)
                        else:
                            recycle()

                        row0 = iota + hl * BT
                        ones = jnp.full((LANES,), 1, jnp.int32)
                        colvs = [ones * (eg * 8 + e) for e in range(8)]

                        @plsc.parallel_loop(0, BT // LANES, unroll=2)
                        def _(b16, slot=slot, colvs=colvs, row0=row0):
                            rowv = row0 + b16 * LANES
                            for e in range(8):
                                vals = plsc.load_gather(buf, [rowv, colvs[e]])
                                stg[slot, e, pl.ds(b16 * LANES, LANES)] = (
                                    vals * SCALE
                                )

                        pltpu.async_copy(stg.at[slot], w_hbm.at[h, eg, bt], osem)

            group(bufa, 0, bufb, True, True)

            @pl.loop(1, NG - 1, step=2)
            def _(g):
                group(bufb, g, bufa, False, True)
                group(bufa, g + 1, bufb, False, True)

            group(bufb, NG - 1, bufa, False, False)

            for slot in range(8):
                pltpu.make_async_copy(
                    stg.at[slot], w_hbm.at[0, 0, bt], osem
                ).wait()

    return kern(table2, idx)


@jax.jit
def kernel(x, table):
    idx = x.reshape(BATCH * HIST).astype(jnp.int32)
    tpad = jnp.pad(table, ((0, 0), (0, 128 - EMBED))).reshape(2 * VOCAB, EMBED)
    w5 = _lookup(tpad, idx)
    return jnp.transpose(w5, (2, 4, 0, 1, 3)).reshape(BATCH, HIST, EMBED)


# speedup vs baseline: 1.2783x; 1.2783x over previous
import dataclasses

import jax
import jax.numpy as jnp
from jax import lax
from jax.experimental import pallas as pl
from jax.experimental.pallas import tpu as pltpu
from jax.experimental.pallas import tpu_sc as plsc

VOCAB = 1000000
EMBED = 64
BATCH = 16384
HIST = 50

LANES = 16
SCALE = 8.0
NW = 32
BT = 128
TILES_PER_W = BATCH // BT // NW
HG = 5
NG = HIST // HG
EG = EMBED // 8

_mesh = plsc.VectorSubcoreMesh(core_axis_name="core", subcore_axis_name="subcore")

_cp = pltpu.CompilerParams(use_tc_tiling_on_sc=False)
if "needs_layout_passes" in pltpu.CompilerParams.__dataclass_fields__:
    _cp = dataclasses.replace(_cp, needs_layout_passes=False)


def _lookup(table2, idx):
    @pl.kernel(
        out_type=jax.ShapeDtypeStruct((HIST, EG, BATCH // BT, 8 * BT), jnp.float32),
        mesh=_mesh,
        scratch_types=[
            pltpu.VMEM((BT * HIST,), jnp.int32),
            pltpu.VMEM((HIST, BT), jnp.int32),
            pltpu.VMEM((HG * BT, EMBED), jnp.float32),
            pltpu.VMEM((HG * BT, EMBED), jnp.float32),
            pltpu.VMEM((2, EMBED * BT), jnp.float32),
            pltpu.SemaphoreType.DMA,
            pltpu.SemaphoreType.DMA,
        ],
        compiler_params=_cp,
    )
    def kern(t_hbm, i_hbm, w_hbm, idxblk, idxt, bufa, bufb, stg, gsem, osem):
        wid = lax.axis_index("subcore") * 2 + lax.axis_index("core")
        iota = lax.iota(jnp.int32, LANES)
        p50 = iota * HIST

        def fire(buf, gi):
            for hl in range(HG):
                pltpu.async_copy(
                    t_hbm.at[idxt.at[gi * HG + hl]],
                    buf.at[pl.ds(hl * BT, BT), :],
                    gsem,
                )

        def drain(buf, gi):
            for hl in range(HG):
                pltpu.make_async_copy(
                    t_hbm.at[idxt.at[gi * HG + hl]],
                    buf.at[pl.ds(hl * BT, BT), :],
                    gsem,
                ).wait()

        @pl.loop(0, TILES_PER_W)
        def _(t):
            bt = wid * TILES_PER_W + t
            pltpu.sync_copy(i_hbm.at[pl.ds(bt * (BT * HIST), BT * HIST)], idxblk)

            @pl.loop(0, HIST)
            def _(h):
                for b16 in range(BT // LANES):
                    v = plsc.load_gather(idxblk, [p50 + (b16 * LANES * HIST + h)])
                    idxt[h, pl.ds(b16 * LANES, LANES)] = v + v

            def group(buf, gi, nbuf, prime, fire_next):
                if prime:
                    fire(buf, gi)
                if fire_next:
                    fire(nbuf, gi + 1)
                drain(buf, gi)

                for hl in range(HG):
                    h = gi * HG + hl
                    slot = hl % 2

                    if not (prime and hl < 2):
                        for k in range(EG):
                            pltpu.make_async_copy(
                                stg.at[slot, pl.ds(k * 1024, 1024)],
                                w_hbm.at[h, k, bt],
                                osem,
                            ).wait()

                    @pl.loop(0, EMBED)
                    def _(e0, hl=hl, slot=slot):
                        t = (iota + e0) & (EMBED - 1)
                        wv0 = (t << 7) + iota
                        for b16 in range(BT // LANES):
                            rowv = iota + (hl * BT + b16 * LANES)
                            vals = plsc.load_gather(buf, [rowv, t])
                            plsc.store_scatter(
                                stg.at[slot], [wv0 + b16 * LANES], vals * SCALE
                            )

                    for eg in range(EG):
                        pltpu.async_copy(
                            stg.at[slot, pl.ds(eg * 1024, 1024)],
                            w_hbm.at[h, eg, bt],
                            osem,
                        )

            group(bufa, 0, bufb, True, True)

            @pl.loop(1, NG - 1, step=2)
            def _(g):
                group(bufb, g, bufa, False, True)
                group(bufa, g + 1, bufb, False, True)

            group(bufb, NG - 1, bufa, False, False)

            for _k in range(2 * EG):
                pltpu.make_async_copy(
                    stg.at[0, pl.ds(0, 1024)], w_hbm.at[0, 0, bt], osem
                ).wait()

    return kern(table2, idx)


@jax.jit
def kernel(x, table):
    idx = x.reshape(BATCH * HIST).astype(jnp.int32)
    tpad = jnp.pad(table, ((0, 0), (0, 128 - EMBED))).reshape(2 * VOCAB, EMBED)
    w5 = _lookup(tpad, idx).reshape(HIST, EG, BATCH // BT, 8, BT)
    return jnp.transpose(w5, (2, 4, 0, 1, 3)).reshape(BATCH, HIST, EMBED)
